# Initial kernel scaffold; baseline (speedup 1.0000x reference)
import jax
import jax.numpy as jnp
from jax.experimental import pallas as pl


def kernel(x, table):
    raise NotImplementedError("write your pallas kernel here")


# speedup vs baseline: 4.5485x; 4.5485x over previous
import functools

import jax
import jax.numpy as jnp
from jax import lax
from jax.experimental import pallas as pl
from jax.experimental.pallas import tpu as pltpu
from jax.experimental.pallas import tpu_sc as plsc

MAX_N = 119
DIM = 64
NC, NS = 2, 16
NW = NC * NS

B = 4096 * 200
PER_W = B // NW
CH = 1024
N_CHUNK = PER_W // CH
SUB = 128
N_SUB = CH // SUB
X_COLS = 128
ROWS_PER_CH = CH // X_COLS


def _emb_body(x_hbm, table_hbm, out_hbm, table_sh, idx_v, rows_v, gat_sem):
    c = lax.axis_index("c")
    s = lax.axis_index("s")
    wid = s * NC + c

    @pl.when(s == 0)
    def _stage():
        pltpu.sync_copy(table_hbm, table_sh.at[pl.ds(1, MAX_N)])

    plsc.subcore_barrier()

    def chunk(ci, carry):
        row0 = wid * (PER_W // X_COLS) + ci * ROWS_PER_CH
        pltpu.sync_copy(x_hbm.at[pl.ds(row0, ROWS_PER_CH)], idx_v)
        copies = [
            pltpu.async_copy(
                table_sh.at[idx_v.at[j]],
                rows_v.at[pl.ds(j * SUB, SUB)],
                gat_sem,
            )
            for j in range(N_SUB)
        ]
        for cp in copies:
            cp.wait()
        base = wid * PER_W + ci * CH
        pltpu.sync_copy(rows_v, out_hbm.at[pl.ds(base, CH)])
        return carry

    lax.fori_loop(0, N_CHUNK, chunk, 0)


@jax.jit
def _emb_call(x2d, table):
    mesh = plsc.VectorSubcoreMesh(core_axis_name="c", subcore_axis_name="s")
    run = pl.kernel(
        _emb_body,
        out_type=jax.ShapeDtypeStruct((B, DIM), jnp.float32),
        mesh=mesh,
        compiler_params=pltpu.CompilerParams(use_tc_tiling_on_sc=False),
        scratch_types=[
            pltpu.VMEM_SHARED((MAX_N + 1, DIM), jnp.float32),
            pltpu.VMEM((ROWS_PER_CH, X_COLS), jnp.int32),
            pltpu.VMEM((CH, DIM), jnp.float32),
            pltpu.SemaphoreType.DMA,
        ],
    )
    return run(x2d, table)


def kernel(x, table):
    x2d = x.reshape(B // X_COLS, X_COLS).astype(jnp.int32)
    out = _emb_call(x2d, table)
    return out.reshape(x.shape[0], x.shape[1], DIM)
